# Initial kernel scaffold; baseline (speedup 1.0000x reference)
import jax
import jax.numpy as jnp
from jax.experimental import pallas as pl


def kernel(features, neigh_idx, num_sample):
    raise NotImplementedError("write your pallas kernel here")


# speedup vs baseline: 1.7223x; 1.7223x over previous
import functools

import jax
import jax.numpy as jnp
from jax import lax
from jax.experimental import pallas as pl
from jax.experimental.pallas import tpu as pltpu
from jax.experimental.pallas import tpu_sc as plsc

D = 128
L = 16
NC = 2
NS = 16
NW = NC * NS
BPW = 320
ROWS_PER_GATHER = 128
OUT_PER_CHUNK = ROWS_PER_GATHER // 32
VPR = D // L


def _make_sc_call(n_nodes, fan_out, scale):
    assert ROWS_PER_GATHER % fan_out == 0
    out_per_chunk = ROWS_PER_GATHER // fan_out
    nchunks = BPW * fan_out // ROWS_PER_GATHER
    assert nchunks % 2 == 0
    npairs = nchunks // 2
    mesh = plsc.VectorSubcoreMesh(core_axis_name="c", subcore_axis_name="s",
                                  num_cores=NC, num_subcores=NS)

    @functools.partial(
        pl.kernel,
        out_type=jax.ShapeDtypeStruct((NW * BPW, D), jnp.float32),
        mesh=mesh,
        scratch_types=[
            pltpu.VMEM((nchunks, ROWS_PER_GATHER), jnp.int32),
            pltpu.VMEM((ROWS_PER_GATHER, D), jnp.float32),
            pltpu.VMEM((ROWS_PER_GATHER, D), jnp.float32),
            pltpu.VMEM((BPW, D), jnp.float32),
            pltpu.SemaphoreType.DMA,
            pltpu.SemaphoreType.DMA,
        ],
    )
    def sc_call(feat_hbm, idx_hbm, out_hbm, idx_v, buf0, buf1, outb,
                sem0, sem1):
        c = lax.axis_index("c")
        s = lax.axis_index("s")
        wid = s * NC + c

        pltpu.sync_copy(idx_hbm.at[wid], idx_v)
        pltpu.async_copy(feat_hbm.at[idx_v.at[0]], buf0, sem0)

        def reduce_chunk(buf, chunk):
            for r in range(out_per_chunk):
                def nbody(n, accs):
                    row = r * fan_out + n
                    return tuple(accs[v] + buf[row, pl.ds(v * L, L)]
                                 for v in range(VPR))
                accs = lax.fori_loop(
                    0, fan_out, nbody,
                    tuple(jnp.zeros((L,), jnp.float32) for _ in range(VPR)))
                orow = chunk * out_per_chunk + r
                for v in range(VPR):
                    outb[orow, pl.ds(v * L, L)] = accs[v] * scale

        def step(t, carry):
            c0 = 2 * t
            c1 = 2 * t + 1
            pltpu.async_copy(feat_hbm.at[idx_v.at[c1]], buf1, sem1)
            pltpu.make_async_copy(feat_hbm.at[idx_v.at[c0]], buf0, sem0).wait()
            reduce_chunk(buf0, c0)

            @pl.when(t + 1 < npairs)
            def _():
                pltpu.async_copy(feat_hbm.at[idx_v.at[c0 + 2]], buf0, sem0)

            pltpu.make_async_copy(feat_hbm.at[idx_v.at[c1]], buf1, sem1).wait()
            reduce_chunk(buf1, c1)
            return carry

        lax.fori_loop(0, npairs, step, 0)
        pltpu.sync_copy(outb, out_hbm.at[pl.ds(wid * BPW, BPW)])

    return sc_call


def kernel(features, neigh_idx, num_sample):
    n_nodes, d = features.shape
    batch, fan_out = neigh_idx.shape
    assert d == D and fan_out == 32
    idx = neigh_idx.astype(jnp.int32)
    pad = NW * BPW - batch
    if pad:
        idx = jnp.pad(idx, ((0, pad), (0, 0)))
    idx3 = idx.reshape(NW, BPW * fan_out // ROWS_PER_GATHER, ROWS_PER_GATHER)
    scale = jnp.float32(1.0 / fan_out)
    sc_call = _make_sc_call(n_nodes, fan_out, scale)
    out = sc_call(features, idx3)
    return out[:batch]


# speedup vs baseline: 1.8025x; 1.0466x over previous
import functools

import jax
import jax.numpy as jnp
from jax import lax
from jax.experimental import pallas as pl
from jax.experimental.pallas import tpu as pltpu
from jax.experimental.pallas import tpu_sc as plsc

D = 128
L = 16
NC = 2
NS = 16
NW = NC * NS
BPW = 320
CBATCH = 64
NCHUNK = BPW // CBATCH
VPR = D // L


def _make_sc_call(fan_out, scale_val):
    mesh = plsc.VectorSubcoreMesh(core_axis_name="c", subcore_axis_name="s",
                                  num_cores=NC, num_subcores=NS)

    @functools.partial(
        pl.kernel,
        out_type=jax.ShapeDtypeStruct((NW * BPW, D), jnp.float32),
        mesh=mesh,
        scratch_types=[
            pltpu.VMEM((fan_out, NCHUNK, CBATCH), jnp.int32),
            pltpu.VMEM((CBATCH, D), jnp.float32),
            pltpu.VMEM((CBATCH, D), jnp.float32),
            pltpu.VMEM((BPW, D), jnp.float32),
            pltpu.SemaphoreType.DMA,
            pltpu.SemaphoreType.DMA,
        ],
    )
    def sc_call(feat_hbm, idx_hbm, out_hbm, idx_v, acc0, acc1, outb,
                sem0, sem1):
        c = lax.axis_index("c")
        s = lax.axis_index("s")
        wid = s * NC + c
        pltpu.sync_copy(idx_hbm.at[wid], idx_v)
        accs = (acc0, acc1)
        sems = (sem0, sem1)
        zvec = jnp.zeros((L,), jnp.float32)

        def zero(acc):
            def zbody(r, carry):
                for v in range(VPR):
                    acc[r, pl.ds(v * L, L)] = zvec
                return carry
            lax.fori_loop(0, CBATCH, zbody, 0)

        def issue(cb, acc, sem):
            def ibody(n, carry):
                pltpu.async_copy(feat_hbm.at[idx_v.at[n, cb]], acc, sem,
                                 add=True)
                return carry
            lax.fori_loop(0, fan_out, ibody, 0)

        def drain(cb, acc, sem):
            def dbody(n, carry):
                pltpu.make_async_copy(feat_hbm.at[idx_v.at[n, cb]], acc,
                                      sem).wait()
                return carry
            lax.fori_loop(0, fan_out, dbody, 0)

        def scale_out(cb, acc):
            def sbody(r, carry):
                for v in range(VPR):
                    outb[cb * CBATCH + r, pl.ds(v * L, L)] = (
                        acc[r, pl.ds(v * L, L)] * scale_val)
                return carry
            lax.fori_loop(0, CBATCH, sbody, 0)

        zero(acc0)
        issue(0, acc0, sem0)
        zero(acc1)
        issue(1, acc1, sem1)
        for cb in range(NCHUNK):
            p = cb % 2
            drain(cb, accs[p], sems[p])
            scale_out(cb, accs[p])
            if cb + 2 < NCHUNK:
                zero(accs[p])
                issue(cb + 2, accs[p], sems[p])
        pltpu.sync_copy(outb, out_hbm.at[pl.ds(wid * BPW, BPW)])

    return sc_call


def kernel(features, neigh_idx, num_sample):
    n_nodes, d = features.shape
    batch, fan_out = neigh_idx.shape
    assert d == D
    idx = neigh_idx.astype(jnp.int32)
    pad = NW * BPW - batch
    if pad:
        idx = jnp.pad(idx, ((0, pad), (0, 0)))
    idx4 = idx.reshape(NW, NCHUNK, CBATCH, fan_out).transpose(0, 3, 1, 2)
    scale = jnp.float32(1.0 / fan_out)
    sc_call = _make_sc_call(fan_out, scale)
    out = sc_call(features, idx4)
    return out[:batch]
